# Initial kernel scaffold; baseline (speedup 1.0000x reference)
import jax
import jax.numpy as jnp
from jax.experimental import pallas as pl


def kernel(x, edge_attr, edge_index, batch, Wres0, Wmsg0, bmsg0, Wupd0, bupd0, Wres1, Wmsg1, bmsg1, Wupd1, bupd1, Wp1, bp1, Wp2, bp2):
    raise NotImplementedError("write your pallas kernel here")


# speedup vs baseline: 3.3826x; 3.3826x over previous
import functools

import jax
import jax.numpy as jnp
from jax import lax
from jax.experimental import pallas as pl
from jax.experimental.pallas import tpu as pltpu
from jax.experimental.pallas import tpu_sc as plsc

N = 10000
E = 320000
D = 128
G = 64

L = 16
NGRP = D // L
K = 80


def _pre_body(h_ref, wm1_ref, wm2_ref, bmsg_ref, a_ref, b_ref):
    h = h_ref[...]
    a_ref[...] = jnp.dot(h, wm1_ref[...], preferred_element_type=jnp.float32)
    b_ref[...] = (jnp.dot(h, wm2_ref[...], preferred_element_type=jnp.float32)
                  + bmsg_ref[...])


def _tc_pre(h, wm1, wm2, bmsg):
    return pl.pallas_call(
        _pre_body,
        out_shape=(jax.ShapeDtypeStruct((N, D), jnp.float32),
                   jax.ShapeDtypeStruct((N, D), jnp.float32)),
    )(h, wm1, wm2, bmsg)


def _post_body(h_ref, p0_ref, p1_ref, wres_ref, wu1_ref, wu2_ref, bupd_ref,
               out_ref):
    h = h_ref[...]
    aggr = p0_ref[...] + p1_ref[...]
    u = (jnp.dot(h, wu1_ref[...], preferred_element_type=jnp.float32)
         + jnp.dot(aggr, wu2_ref[...], preferred_element_type=jnp.float32)
         + bupd_ref[...])
    out_ref[...] = (jnp.dot(h, wres_ref[...], preferred_element_type=jnp.float32)
                    + jnp.maximum(u, 0.0))


def _tc_post(h, p0, p1, wres, wu1, wu2, bupd):
    return pl.pallas_call(
        _post_body,
        out_shape=jax.ShapeDtypeStruct((N, D), jnp.float32),
    )(h, p0, p1, wres, wu1, wu2, bupd)


def _readout_body(h_ref, batch_ref, wp1_ref, bp1_ref, wp2_ref, bp2_ref,
                  out_ref):
    h = h_ref[...]
    b = batch_ref[0:1, :]
    gids = lax.broadcasted_iota(jnp.int32, (G, N), 0)
    onehot = (gids == b).astype(jnp.float32)
    g = jnp.dot(onehot, h, preferred_element_type=jnp.float32)
    t = jnp.maximum(
        jnp.dot(g, wp1_ref[...], preferred_element_type=jnp.float32)
        + bp1_ref[...], 0.0)
    out_ref[...] = (jnp.dot(t, wp2_ref[...], preferred_element_type=jnp.float32)
                    + bp2_ref[...])


def _tc_readout(h, batch8, wp1, bp1, wp2pad, bp2b):
    return pl.pallas_call(
        _readout_body,
        out_shape=jax.ShapeDtypeStruct((G, D), jnp.float32),
    )(h, batch8, wp1, bp1, wp2pad, bp2b)


def _sc_edge_body(a_hbm, b_hbm, wv_hbm, row_hbm, col_hbm, ea_hbm, out_hbm,
                  idxr_v, idxc_v, ea_v, a_v, b_v, m_v, wv_v, z_v, aggr_sh,
                  sem_a, sem_b):
    nc = 2
    ns = 16
    c = lax.axis_index("c")
    s = lax.axis_index("s")
    w = s * nc + c
    epw = E // (nc * ns)
    zrows = 80
    nchunk = N // zrows

    def _zero_row(i, _):
        for gi in range(NGRP):
            z_v[i, pl.ds(gi * L, L)] = jnp.zeros((L,), jnp.float32)
        return 0
    lax.fori_loop(0, zrows, _zero_row, 0)
    for t in range(pl.cdiv(nchunk, ns)):
        cid = t * ns + s
        @pl.when(cid < nchunk)
        def _():
            pltpu.sync_copy(z_v, aggr_sh.at[pl.ds(cid * zrows, zrows)])
    plsc.subcore_barrier()

    pltpu.sync_copy(wv_hbm, wv_v)
    wvs = [wv_v[pl.ds(gi * L, L)] for gi in range(NGRP)]

    base = w * epw

    def _block(t, _):
        off = base + t * K
        pltpu.sync_copy(row_hbm.at[pl.ds(off, K)], idxr_v)
        pltpu.sync_copy(col_hbm.at[pl.ds(off, K)], idxc_v)
        pltpu.sync_copy(ea_hbm.at[pl.ds(off * L, K * L)], ea_v)
        cp_a = pltpu.async_copy(a_hbm.at[idxr_v], a_v, sem_a)
        cp_b = pltpu.async_copy(b_hbm.at[idxc_v], b_v, sem_b)
        cp_a.wait()
        cp_b.wait()

        def _edge(j, _):
            ea_s = ea_v[pl.ds(j * L, L)]
            for gi in range(NGRP):
                sl = pl.ds(gi * L, L)
                m = a_v[j, sl] + b_v[j, sl] + ea_s * wvs[gi]
                m_v[j, sl] = jnp.maximum(m, 0.0)
            return 0
        lax.fori_loop(0, K, _edge, 0)

        pltpu.sync_copy(m_v, aggr_sh.at[idxc_v], add=True)
        return 0

    lax.fori_loop(0, epw // K, _block, 0)
    plsc.subcore_barrier()

    for t in range(pl.cdiv(nchunk, ns)):
        cid = t * ns + s
        @pl.when(cid < nchunk)
        def _():
            r0 = cid * zrows
            pltpu.sync_copy(aggr_sh.at[pl.ds(r0, zrows)], z_v)
            pltpu.sync_copy(z_v, out_hbm.at[c, pl.ds(r0, zrows)])


def _sc_edge(a, b, wv, row, col, ea):
    mesh = plsc.VectorSubcoreMesh(core_axis_name="c", subcore_axis_name="s")
    fn = functools.partial(
        pl.kernel,
        out_type=jax.ShapeDtypeStruct((2, N, D), jnp.float32),
        mesh=mesh,
        scratch_types=[
            pltpu.VMEM((K,), jnp.int32),
            pltpu.VMEM((K,), jnp.int32),
            pltpu.VMEM((K * L,), jnp.float32),
            pltpu.VMEM((K, D), jnp.float32),
            pltpu.VMEM((K, D), jnp.float32),
            pltpu.VMEM((K, D), jnp.float32),
            pltpu.VMEM((D,), jnp.float32),
            pltpu.VMEM((80, D), jnp.float32),
            pltpu.VMEM_SHARED((N, D), jnp.float32),
            pltpu.SemaphoreType.DMA,
            pltpu.SemaphoreType.DMA,
        ],
    )(_sc_edge_body)
    return fn(a, b, wv, row, col, ea)


def kernel(x, edge_attr, edge_index, batch,
           Wres0, Wmsg0, bmsg0, Wupd0, bupd0,
           Wres1, Wmsg1, bmsg1, Wupd1, bupd1,
           Wp1, bp1, Wp2, bp2):
    row = edge_index[0]
    col = edge_index[1]
    ea = jnp.broadcast_to(edge_attr.reshape(E, 1), (E, L)).reshape(E * L)

    h = x
    for (Wres, Wmsg, bmsg, Wupd, bupd) in (
            (Wres0, Wmsg0, bmsg0, Wupd0, bupd0),
            (Wres1, Wmsg1, bmsg1, Wupd1, bupd1)):
        wm1 = Wmsg[:D]
        wm2 = Wmsg[D:2 * D]
        wv = Wmsg[2 * D]
        wu1 = Wupd[:D]
        wu2 = Wupd[D:]
        a, b = _tc_pre(h, wm1, wm2, bmsg.reshape(1, D))
        parts = _sc_edge(a, b, wv, row, col, ea)
        h = _tc_post(h, parts[0], parts[1], Wres, wu1, wu2, bupd.reshape(1, D))

    batch8 = jnp.broadcast_to(batch.reshape(1, N), (8, N))
    wp2pad = jnp.pad(Wp2, ((0, 0), (0, D - 1)))
    bp2b = jnp.broadcast_to(bp2.reshape(1, 1), (1, D))
    out = _tc_readout(h, batch8, Wp1, bp1.reshape(1, D), wp2pad, bp2b)
    return out[:, :1]
